# Initial kernel scaffold; baseline (speedup 1.0000x reference)
import jax
import jax.numpy as jnp
from jax.experimental import pallas as pl


def kernel(voxel_importance, voxel_coords, voxels, voxel_num_points):
    raise NotImplementedError("write your pallas kernel here")


# speedup vs baseline: 7.5996x; 7.5996x over previous
import functools

import jax
import jax.numpy as jnp
from jax import lax
from jax.experimental import pallas as pl
from jax.experimental.pallas import tpu as pltpu
from jax.experimental.pallas import tpu_sc as plsc

N = 150000
KEEP = 75000

_NC = 2
_NS = 16
_NW = _NC * _NS

_NBITS = (11, 11, 10)
_SHIFTS = (0, 11, 22)
_NBINS = 2048
_ROWS = 74
_CHUNK = 9376
_CHUNK_LAST = 9360
_SLACK = _NS * 128
_SBUF = N + _SLACK


def _sortable_u32(f):
    b = lax.bitcast_convert_type(f, jnp.uint32)
    sgn = b >> jnp.uint32(31)
    return b ^ (jnp.uint32(0x80000000) | (sgn * jnp.uint32(0x7FFFFFFF)))


def _sort_body(imp_hbm, out_hbm,
               key_v, idx_v, pos_v, hist_v, tots_v, acc_v, g_v,
               key_a, idx_a, key_b, idx_b, tot_sh,
               sem):
    core = lax.axis_index("c")
    t = lax.axis_index("s")
    lanes = lax.iota(jnp.int32, 16)
    lanes16 = lanes * 16
    base = t * _CHUNK
    is_last = t == _NS - 1
    ones = jnp.ones((16,), jnp.int32)
    rcol = jnp.where(is_last, _CHUNK_LAST // 16, _CHUNK // 16)
    lanesr = lanes * rcol

    @pl.when(core == 0)
    def _sort():
        @pl.when(is_last)
        def _():
            pltpu.sync_copy(imp_hbm.at[pl.ds(base, _CHUNK_LAST)],
                            key_v.at[pl.ds(0, _CHUNK_LAST)])

        @pl.when(jnp.logical_not(is_last))
        def _():
            pltpu.sync_copy(imp_hbm.at[pl.ds(base, _CHUNK)],
                            key_v.at[pl.ds(0, _CHUNK)])

        def seed_row(r, _):
            for u in range(8):
                off = r * 128 + u * 16
                key_v[pl.ds(off, 16)] = lax.bitcast_convert_type(
                    _sortable_u32(key_v[pl.ds(off, 16)]), jnp.float32)
                idx_v[pl.ds(off, 16)] = base + off + lanes
            return _
        lax.fori_loop(0, _ROWS, seed_row, 0)

        for p in range(3):
            shift = jnp.uint32(_SHIFTS[p])
            mask = jnp.uint32((1 << _NBITS[p]) - 1)
            src_k, src_i = (key_a, idx_a) if p == 1 else (key_b, idx_b)
            dst_k, dst_i = (key_b, idx_b) if p == 1 else (key_a, idx_a)
            last_pass = p == 2

            if p > 0:
                @pl.when(is_last)
                def _():
                    pltpu.sync_copy(src_k.at[pl.ds(base, _CHUNK_LAST)],
                                    key_v.at[pl.ds(0, _CHUNK_LAST)])
                    pltpu.sync_copy(src_i.at[pl.ds(base, _CHUNK_LAST)],
                                    idx_v.at[pl.ds(0, _CHUNK_LAST)])

                @pl.when(jnp.logical_not(is_last))
                def _():
                    pltpu.sync_copy(src_k.at[pl.ds(base, _CHUNK)],
                                    key_v.at[pl.ds(0, _CHUNK)])
                    pltpu.sync_copy(src_i.at[pl.ds(base, _CHUNK)],
                                    idx_v.at[pl.ds(0, _CHUNK)])

            def digits(s):
                kw = plsc.load_gather(key_v, [lanesr + s])
                kb = lax.bitcast_convert_type(kw, jnp.uint32)
                d = (kb >> shift) & mask
                return d.astype(jnp.int32) * 16 + lanes

            def zero_row(r, _):
                for u in range(8):
                    hist_v[pl.ds(r * 128 + u * 16, 16)] = jnp.zeros(
                        (16,), jnp.int32)
                return _
            lax.fori_loop(0, _NBINS * 16 // 128, zero_row, 0)

            def count_step(s, _):
                fl = digits(s)
                plsc.store_scatter(hist_v, [fl],
                                   plsc.load_gather(hist_v, [fl]) + ones)
                return _
            lax.fori_loop(0, rcol, count_step, 0)

            def tot_group(bv, _):
                s = jnp.zeros((16,), jnp.int32)
                for l in range(16):
                    s = s + plsc.load_gather(
                        hist_v, [bv * 256 + lanes16 + l])
                g_v[pl.ds(bv * 16, 16)] = s
                return _
            lax.fori_loop(0, _NBINS // 16, tot_group, 0)
            pltpu.sync_copy(g_v, tot_sh.at[pl.ds(t * _NBINS, _NBINS)])
            plsc.subcore_barrier()

            def zacc(bv, _):
                acc_v[pl.ds(bv * 16, 16)] = jnp.zeros((16,), jnp.int32)
                g_v[pl.ds(bv * 16, 16)] = jnp.zeros((16,), jnp.int32)
                return _
            lax.fori_loop(0, _NBINS // 16, zacc, 0)
            for chunk in range(4):
                pltpu.sync_copy(
                    tot_sh.at[pl.ds(chunk * 4 * _NBINS, 4 * _NBINS)],
                    tots_v)

                def accgrp(bv, _):
                    at = acc_v[pl.ds(bv * 16, 16)]
                    ap = g_v[pl.ds(bv * 16, 16)]
                    for tl in range(4):
                        tt = chunk * 4 + tl
                        row = tots_v[pl.ds(tl * _NBINS + bv * 16, 16)]
                        at = at + row
                        ap = ap + jnp.where(
                            jnp.full((16,), tt, jnp.int32) < t, row,
                            jnp.zeros((16,), jnp.int32))
                    acc_v[pl.ds(bv * 16, 16)] = at
                    g_v[pl.ds(bv * 16, 16)] = ap
                    return _
                lax.fori_loop(0, _NBINS // 16, accgrp, 0)

            def goff_group(bv, carry):
                tot = acc_v[pl.ds(bv * 16, 16)]
                excl = carry + jnp.cumsum(tot) - tot
                g_v[pl.ds(bv * 16, 16)] = g_v[pl.ds(bv * 16, 16)] + excl
                return carry + jnp.sum(tot)
            lax.fori_loop(0, _NBINS // 16, goff_group, jnp.int32(0))

            def sinit(b, _):
                hv = hist_v[pl.ds(b * 16, 16)]
                gb = plsc.load_gather(g_v, [jnp.zeros((16,), jnp.int32) + b])
                hist_v[pl.ds(b * 16, 16)] = gb + jnp.cumsum(hv) - hv
                return _
            lax.fori_loop(0, _NBINS, sinit, 0)

            def rank_step(s, _):
                fl = digits(s)
                b0 = plsc.load_gather(hist_v, [fl])
                plsc.store_scatter(hist_v, [fl], b0 + ones)
                q = lanesr + s
                plsc.store_scatter(pos_v, [q >> 7, q & 127], b0)
                return _
            lax.fori_loop(0, rcol, rank_step, 0)

            for u in range(8):
                dummy = (jnp.zeros((16,), jnp.int32) + (N + u * 16)
                         + t * 128 + lanes)
                if u >= 2:
                    plsc.store_scatter(
                        pos_v, [jnp.zeros((16,), jnp.int32) + 73,
                                lanes + u * 16], dummy)
                elif u == 1:
                    @pl.when(is_last)
                    def _():
                        plsc.store_scatter(
                            pos_v, [jnp.zeros((16,), jnp.int32) + 73,
                                    lanes + u * 16], dummy)

            def scat_row(r, _):
                if not last_pass:
                    pltpu.sync_copy(key_v.at[pl.ds(r * 128, 128)],
                                    dst_k.at[pos_v.at[r]])
                pltpu.sync_copy(idx_v.at[pl.ds(r * 128, 128)],
                                dst_i.at[pos_v.at[r]])
                return _
            lax.fori_loop(0, _ROWS, scat_row, 0)
            plsc.subcore_barrier()

        @pl.when(is_last)
        def _():
            pltpu.sync_copy(idx_a.at[pl.ds(base, _CHUNK_LAST)],
                            idx_v.at[pl.ds(0, _CHUNK_LAST)])
            pltpu.sync_copy(idx_v.at[pl.ds(0, _CHUNK_LAST)],
                            out_hbm.at[pl.ds(base, _CHUNK_LAST)])

        @pl.when(jnp.logical_not(is_last))
        def _():
            pltpu.sync_copy(idx_a.at[pl.ds(base, _CHUNK)],
                            idx_v.at[pl.ds(0, _CHUNK)])
            pltpu.sync_copy(idx_v.at[pl.ds(0, _CHUNK)],
                            out_hbm.at[pl.ds(base, _CHUNK)])


def _make_sort():
    return functools.partial(
        pl.kernel,
        out_type=jax.ShapeDtypeStruct((N,), jnp.int32),
        mesh=plsc.VectorSubcoreMesh(core_axis_name="c", subcore_axis_name="s"),
        compiler_params=pltpu.CompilerParams(needs_layout_passes=False),
        scratch_types=[
            pltpu.VMEM((_ROWS * 128,), jnp.float32),
            pltpu.VMEM((_ROWS * 128,), jnp.int32),
            pltpu.VMEM((_ROWS, 128), jnp.int32),
            pltpu.VMEM((_NBINS * 16,), jnp.int32),
            pltpu.VMEM((4 * _NBINS,), jnp.int32),
            pltpu.VMEM((_NBINS,), jnp.int32),
            pltpu.VMEM((_NBINS,), jnp.int32),
            pltpu.VMEM_SHARED((_SBUF,), jnp.float32),
            pltpu.VMEM_SHARED((_SBUF,), jnp.int32),
            pltpu.VMEM_SHARED((_SBUF,), jnp.float32),
            pltpu.VMEM_SHARED((_SBUF,), jnp.int32),
            pltpu.VMEM_SHARED((_NS * _NBINS,), jnp.int32),
            pltpu.SemaphoreType.DMA,
        ],
    )(_sort_body)


_FULLW = 585
_LASTW_ROWS = 120


def _gather_body(keep_hbm, cflat_hbm, vox_hbm, nump_hbm,
                 cflat_out, vox_out, nump_out,
                 idx_v, vox_v, np_v, cidx_v, col_v, crd_v,
                 sem):
    c = lax.axis_index("c")
    s = lax.axis_index("s")
    wid = s * _NC + c
    lanes = lax.iota(jnp.int32, 16)

    def do_window(off, nrows):
        idx_r = idx_v.at[pl.ds(0, nrows)] if nrows != 128 else idx_v
        pltpu.sync_copy(keep_hbm.at[pl.ds(off, nrows)], idx_r)
        pltpu.async_copy(vox_hbm.at[idx_r],
                         vox_v.at[pl.ds(0, nrows)], sem).wait()
        pltpu.sync_copy(vox_v.at[pl.ds(0, nrows)],
                        vox_out.at[pl.ds(off, nrows)])
        pltpu.async_copy(nump_hbm.at[idx_r],
                         np_v.at[pl.ds(0, nrows)], sem).wait()
        pltpu.sync_copy(np_v.at[pl.ds(0, nrows)],
                        nump_out.at[pl.ds(off, nrows)])
        nsub = (nrows + 15) // 16
        tail_valid = nrows - (nsub - 1) * 16
        for cc in range(4):
            def cidx_row(j, _):
                iv = idx_v[pl.ds(j * 16, 16)]
                cidx_v[pl.ds(j * 16, 16)] = iv * 4 + cc
                return _
            lax.fori_loop(0, nsub - 1, cidx_row, 0)
            jt = nsub - 1
            ivt = idx_v[pl.ds(jt * 16, 16)]
            lmask = lanes < tail_valid
            cidx_v[pl.ds(jt * 16, 16)] = jnp.where(
                lmask, ivt * 4 + cc, jnp.zeros((16,), jnp.int32))
            pltpu.async_copy(cflat_hbm.at[cidx_v.at[pl.ds(0, nrows)]],
                             col_v.at[pl.ds(0, nrows)], sem).wait()

            def ileave_row(j, _):
                plsc.store_scatter(crd_v, [j * 64 + lanes * 4 + cc],
                                   col_v[pl.ds(j * 16, 16)])
                return _
            lax.fori_loop(0, nsub - 1, ileave_row, 0)
            plsc.store_scatter(crd_v, [jt * 64 + lanes * 4 + cc],
                               col_v[pl.ds(jt * 16, 16)], mask=lmask)
        pltpu.sync_copy(crd_v.at[pl.ds(0, nrows * 4)],
                        cflat_out.at[pl.ds(off * 4, nrows * 4)])

    nfull = jnp.where(wid < 9, 19, 18)

    def wbody(k, _):
        do_window((wid + 32 * k) * 128, 128)
        return _
    lax.fori_loop(0, nfull, wbody, 0)

    @pl.when(wid == 9)
    def _():
        do_window(_FULLW * 128, _LASTW_ROWS)


def _make_gather():
    return functools.partial(
        pl.kernel,
        out_type=(
            jax.ShapeDtypeStruct((KEEP * 4,), jnp.int32),
            jax.ShapeDtypeStruct((KEEP, 128), jnp.float32),
            jax.ShapeDtypeStruct((KEEP,), jnp.int32),
        ),
        mesh=plsc.VectorSubcoreMesh(core_axis_name="c", subcore_axis_name="s"),
        compiler_params=pltpu.CompilerParams(needs_layout_passes=False),
        scratch_types=[
            pltpu.VMEM((128,), jnp.int32),
            pltpu.VMEM((128, 128), jnp.float32),
            pltpu.VMEM((128,), jnp.int32),
            pltpu.VMEM((128,), jnp.int32),
            pltpu.VMEM((128,), jnp.int32),
            pltpu.VMEM((512,), jnp.int32),
            pltpu.SemaphoreType.DMA,
        ],
    )(_gather_body)


def kernel(voxel_importance, voxel_coords, voxels, voxel_num_points):
    order = _make_sort()(voxel_importance)
    keep = lax.slice(order, (KEEP,), (N,))
    vox2d = voxels.reshape(N, 128)
    cflat = voxel_coords.reshape(N * 4)
    cflat_o, vox_o, nump_o = _make_gather()(
        keep, cflat, vox2d, voxel_num_points)
    return (cflat_o.reshape(KEEP, 4), vox_o.reshape(KEEP, 32, 4), nump_o)


# speedup vs baseline: 8.6059x; 1.1324x over previous
import functools

import jax
import jax.numpy as jnp
from jax import lax
from jax.experimental import pallas as pl
from jax.experimental.pallas import tpu as pltpu
from jax.experimental.pallas import tpu_sc as plsc

N = 150000
KEEP = 75000

_NC = 2
_NS = 16
_NW = _NC * _NS

_NBITS = (11, 11, 10)
_SHIFTS = (0, 11, 22)
_NBINS = 2048
_ROWS = 74
_CHUNK = 9376
_CHUNK_LAST = 9360
_SLACK = _NS * 128
_SBUF = N + _SLACK


def _sortable_u32(f):
    b = lax.bitcast_convert_type(f, jnp.uint32)
    sgn = b >> jnp.uint32(31)
    return b ^ (jnp.uint32(0x80000000) | (sgn * jnp.uint32(0x7FFFFFFF)))


def _sort_body(imp_hbm, out_hbm,
               key_v, idx_v, pos_v, hist_v, tots_v, acc_v, g_v,
               key_a, idx_a, key_b, idx_b, tot_sh,
               sem):
    core = lax.axis_index("c")
    t = lax.axis_index("s")
    lanes = lax.iota(jnp.int32, 16)
    lanes16 = lanes * 16
    base = t * _CHUNK
    is_last = t == _NS - 1
    ones = jnp.ones((16,), jnp.int32)
    rcol = jnp.where(is_last, _CHUNK_LAST // 16, _CHUNK // 16)
    lanesr = lanes * rcol

    @pl.when(core == 0)
    def _sort():
        @pl.when(is_last)
        def _():
            pltpu.sync_copy(imp_hbm.at[pl.ds(base, _CHUNK_LAST)],
                            key_v.at[pl.ds(0, _CHUNK_LAST)])

        @pl.when(jnp.logical_not(is_last))
        def _():
            pltpu.sync_copy(imp_hbm.at[pl.ds(base, _CHUNK)],
                            key_v.at[pl.ds(0, _CHUNK)])

        def seed_row(r, _):
            for u in range(8):
                off = r * 128 + u * 16
                key_v[pl.ds(off, 16)] = lax.bitcast_convert_type(
                    _sortable_u32(key_v[pl.ds(off, 16)]), jnp.float32)
                idx_v[pl.ds(off, 16)] = base + off + lanes
            return _
        lax.fori_loop(0, _ROWS, seed_row, 0)

        for p in range(3):
            shift = jnp.uint32(_SHIFTS[p])
            mask = jnp.uint32((1 << _NBITS[p]) - 1)
            src_k, src_i = (key_a, idx_a) if p == 1 else (key_b, idx_b)
            dst_k, dst_i = (key_b, idx_b) if p == 1 else (key_a, idx_a)
            last_pass = p == 2

            if p > 0:
                @pl.when(is_last)
                def _():
                    pltpu.sync_copy(src_k.at[pl.ds(base, _CHUNK_LAST)],
                                    key_v.at[pl.ds(0, _CHUNK_LAST)])
                    pltpu.sync_copy(src_i.at[pl.ds(base, _CHUNK_LAST)],
                                    idx_v.at[pl.ds(0, _CHUNK_LAST)])

                @pl.when(jnp.logical_not(is_last))
                def _():
                    pltpu.sync_copy(src_k.at[pl.ds(base, _CHUNK)],
                                    key_v.at[pl.ds(0, _CHUNK)])
                    pltpu.sync_copy(src_i.at[pl.ds(base, _CHUNK)],
                                    idx_v.at[pl.ds(0, _CHUNK)])

            def digits(s):
                kw = plsc.load_gather(key_v, [lanesr + s])
                kb = lax.bitcast_convert_type(kw, jnp.uint32)
                d = (kb >> shift) & mask
                return d.astype(jnp.int32) + lanes * _NBINS

            def zero_row(r, _):
                for u in range(8):
                    hist_v[pl.ds(r * 128 + u * 16, 16)] = jnp.zeros(
                        (16,), jnp.int32)
                return _
            lax.fori_loop(0, _NBINS * 16 // 128, zero_row, 0)

            def count_step(s, _):
                fl = digits(s)
                plsc.store_scatter(hist_v, [fl],
                                   plsc.load_gather(hist_v, [fl]) + ones)
                return _
            lax.fori_loop(0, rcol, count_step, 0)

            def tot_group(bv, _):
                s = jnp.zeros((16,), jnp.int32)
                for l in range(16):
                    s = s + hist_v[pl.ds(l * _NBINS + bv * 16, 16)]
                g_v[pl.ds(bv * 16, 16)] = s
                return _
            lax.fori_loop(0, _NBINS // 16, tot_group, 0)
            pltpu.sync_copy(g_v, tot_sh.at[pl.ds(t * _NBINS, _NBINS)])
            plsc.subcore_barrier()

            def zacc(bv, _):
                acc_v[pl.ds(bv * 16, 16)] = jnp.zeros((16,), jnp.int32)
                g_v[pl.ds(bv * 16, 16)] = jnp.zeros((16,), jnp.int32)
                return _
            lax.fori_loop(0, _NBINS // 16, zacc, 0)
            for chunk in range(4):
                pltpu.sync_copy(
                    tot_sh.at[pl.ds(chunk * 4 * _NBINS, 4 * _NBINS)],
                    tots_v)

                def accgrp(bv, _):
                    at = acc_v[pl.ds(bv * 16, 16)]
                    ap = g_v[pl.ds(bv * 16, 16)]
                    for tl in range(4):
                        tt = chunk * 4 + tl
                        row = tots_v[pl.ds(tl * _NBINS + bv * 16, 16)]
                        at = at + row
                        ap = ap + jnp.where(
                            jnp.full((16,), tt, jnp.int32) < t, row,
                            jnp.zeros((16,), jnp.int32))
                    acc_v[pl.ds(bv * 16, 16)] = at
                    g_v[pl.ds(bv * 16, 16)] = ap
                    return _
                lax.fori_loop(0, _NBINS // 16, accgrp, 0)

            def goff_group(bv, carry):
                tot = acc_v[pl.ds(bv * 16, 16)]
                excl = carry + jnp.cumsum(tot) - tot
                g_v[pl.ds(bv * 16, 16)] = g_v[pl.ds(bv * 16, 16)] + excl
                return carry + jnp.sum(tot)
            lax.fori_loop(0, _NBINS // 16, goff_group, jnp.int32(0))

            def sinit(bv, _):
                acc = g_v[pl.ds(bv * 16, 16)]
                for l in range(16):
                    hv = hist_v[pl.ds(l * _NBINS + bv * 16, 16)]
                    hist_v[pl.ds(l * _NBINS + bv * 16, 16)] = acc
                    acc = acc + hv
                return _
            lax.fori_loop(0, _NBINS // 16, sinit, 0)

            def rank_step(s, _):
                fl = digits(s)
                b0 = plsc.load_gather(hist_v, [fl])
                plsc.store_scatter(hist_v, [fl], b0 + ones)
                q = lanesr + s
                plsc.store_scatter(pos_v, [q >> 7, q & 127], b0)
                return _
            lax.fori_loop(0, rcol, rank_step, 0)

            for u in range(8):
                dummy = (jnp.zeros((16,), jnp.int32) + (N + u * 16)
                         + t * 128 + lanes)
                if u >= 2:
                    plsc.store_scatter(
                        pos_v, [jnp.zeros((16,), jnp.int32) + 73,
                                lanes + u * 16], dummy)
                elif u == 1:
                    @pl.when(is_last)
                    def _():
                        plsc.store_scatter(
                            pos_v, [jnp.zeros((16,), jnp.int32) + 73,
                                    lanes + u * 16], dummy)

            def scat_row(r, _):
                if not last_pass:
                    pltpu.sync_copy(key_v.at[pl.ds(r * 128, 128)],
                                    dst_k.at[pos_v.at[r]])
                pltpu.sync_copy(idx_v.at[pl.ds(r * 128, 128)],
                                dst_i.at[pos_v.at[r]])
                return _
            lax.fori_loop(0, _ROWS, scat_row, 0)
            plsc.subcore_barrier()

        @pl.when(is_last)
        def _():
            pltpu.sync_copy(idx_a.at[pl.ds(base, _CHUNK_LAST)],
                            idx_v.at[pl.ds(0, _CHUNK_LAST)])
            pltpu.sync_copy(idx_v.at[pl.ds(0, _CHUNK_LAST)],
                            out_hbm.at[pl.ds(base, _CHUNK_LAST)])

        @pl.when(jnp.logical_not(is_last))
        def _():
            pltpu.sync_copy(idx_a.at[pl.ds(base, _CHUNK)],
                            idx_v.at[pl.ds(0, _CHUNK)])
            pltpu.sync_copy(idx_v.at[pl.ds(0, _CHUNK)],
                            out_hbm.at[pl.ds(base, _CHUNK)])


def _make_sort():
    return functools.partial(
        pl.kernel,
        out_type=jax.ShapeDtypeStruct((N,), jnp.int32),
        mesh=plsc.VectorSubcoreMesh(core_axis_name="c", subcore_axis_name="s"),
        compiler_params=pltpu.CompilerParams(needs_layout_passes=False),
        scratch_types=[
            pltpu.VMEM((_ROWS * 128,), jnp.float32),
            pltpu.VMEM((_ROWS * 128,), jnp.int32),
            pltpu.VMEM((_ROWS, 128), jnp.int32),
            pltpu.VMEM((_NBINS * 16,), jnp.int32),
            pltpu.VMEM((4 * _NBINS,), jnp.int32),
            pltpu.VMEM((_NBINS,), jnp.int32),
            pltpu.VMEM((_NBINS,), jnp.int32),
            pltpu.VMEM_SHARED((_SBUF,), jnp.float32),
            pltpu.VMEM_SHARED((_SBUF,), jnp.int32),
            pltpu.VMEM_SHARED((_SBUF,), jnp.float32),
            pltpu.VMEM_SHARED((_SBUF,), jnp.int32),
            pltpu.VMEM_SHARED((_NS * _NBINS,), jnp.int32),
            pltpu.SemaphoreType.DMA,
        ],
    )(_sort_body)


_FULLW = 585
_LASTW_ROWS = 120


def _gather_body(keep_hbm, cflat_hbm, vox_hbm, nump_hbm,
                 cflat_out, vox_out, nump_out,
                 idx_v, vox_v, np_v, cidx_v, col_v, crd_v,
                 sem_v, sem_n, sem_c):
    c = lax.axis_index("c")
    s = lax.axis_index("s")
    wid = s * _NC + c
    lanes = lax.iota(jnp.int32, 16)

    def do_window(off, nrows):
        idx_r = idx_v.at[pl.ds(0, nrows)] if nrows != 128 else idx_v
        pltpu.sync_copy(keep_hbm.at[pl.ds(off, nrows)], idx_r)
        nsub = (nrows + 15) // 16
        tail_valid = nrows - (nsub - 1) * 16
        lmask = lanes < tail_valid
        for cc in range(4):
            def cidx_row(j, _):
                iv = idx_v[pl.ds(j * 16, 16)]
                cidx_v[pl.ds(cc * 128 + j * 16, 16)] = iv * 4 + cc
                return _
            lax.fori_loop(0, nsub - 1, cidx_row, 0)
            jt = nsub - 1
            ivt = idx_v[pl.ds(jt * 16, 16)]
            cidx_v[pl.ds(cc * 128 + jt * 16, 16)] = jnp.where(
                lmask, ivt * 4 + cc, jnp.zeros((16,), jnp.int32))
        dv = pltpu.async_copy(vox_hbm.at[idx_r],
                              vox_v.at[pl.ds(0, nrows)], sem_v)
        dn = pltpu.async_copy(nump_hbm.at[idx_r],
                              np_v.at[pl.ds(0, nrows)], sem_n)
        dc = [pltpu.async_copy(
                  cflat_hbm.at[cidx_v.at[pl.ds(cc * 128, nrows)]],
                  col_v.at[pl.ds(cc * 128, nrows)], sem_c)
              for cc in range(4)]
        dv.wait()
        pltpu.sync_copy(vox_v.at[pl.ds(0, nrows)],
                        vox_out.at[pl.ds(off, nrows)])
        dn.wait()
        pltpu.sync_copy(np_v.at[pl.ds(0, nrows)],
                        nump_out.at[pl.ds(off, nrows)])
        for d in dc:
            d.wait()
        for cc in range(4):
            def ileave_row(j, _):
                plsc.store_scatter(crd_v, [j * 64 + lanes * 4 + cc],
                                   col_v[pl.ds(cc * 128 + j * 16, 16)])
                return _
            lax.fori_loop(0, nsub - 1, ileave_row, 0)
            jt = nsub - 1
            plsc.store_scatter(crd_v, [jt * 64 + lanes * 4 + cc],
                               col_v[pl.ds(cc * 128 + jt * 16, 16)],
                               mask=lmask)
        pltpu.sync_copy(crd_v.at[pl.ds(0, nrows * 4)],
                        cflat_out.at[pl.ds(off * 4, nrows * 4)])

    nfull = jnp.where(wid < 9, 19, 18)

    def wbody(k, _):
        do_window((wid + 32 * k) * 128, 128)
        return _
    lax.fori_loop(0, nfull, wbody, 0)

    @pl.when(wid == 9)
    def _():
        do_window(_FULLW * 128, _LASTW_ROWS)


def _make_gather():
    return functools.partial(
        pl.kernel,
        out_type=(
            jax.ShapeDtypeStruct((KEEP * 4,), jnp.int32),
            jax.ShapeDtypeStruct((KEEP, 128), jnp.float32),
            jax.ShapeDtypeStruct((KEEP,), jnp.int32),
        ),
        mesh=plsc.VectorSubcoreMesh(core_axis_name="c", subcore_axis_name="s"),
        compiler_params=pltpu.CompilerParams(needs_layout_passes=False),
        scratch_types=[
            pltpu.VMEM((128,), jnp.int32),
            pltpu.VMEM((128, 128), jnp.float32),
            pltpu.VMEM((128,), jnp.int32),
            pltpu.VMEM((512,), jnp.int32),
            pltpu.VMEM((512,), jnp.int32),
            pltpu.VMEM((512,), jnp.int32),
            pltpu.SemaphoreType.DMA,
            pltpu.SemaphoreType.DMA,
            pltpu.SemaphoreType.DMA,
        ],
    )(_gather_body)


def kernel(voxel_importance, voxel_coords, voxels, voxel_num_points):
    order = _make_sort()(voxel_importance)
    keep = lax.slice(order, (KEEP,), (N,))
    cflat = voxel_coords.reshape(N * 4)
    vox2d = voxels.reshape(N, 128)
    cflat_o, vox_o, nump_o = _make_gather()(
        keep, cflat, vox2d, voxel_num_points)
    return (cflat_o.reshape(KEEP, 4), vox_o.reshape(KEEP, 32, 4), nump_o)
